# Initial kernel scaffold; baseline (speedup 1.0000x reference)
import jax
import jax.numpy as jnp
from jax.experimental import pallas as pl


def kernel(x, edge_index, batch, W1, b1, W2, b2, Wb, bb):
    raise NotImplementedError("write your pallas kernel here")


# speedup vs baseline: 9.1661x; 9.1661x over previous
import functools

import jax
import jax.numpy as jnp
from jax import lax
from jax.experimental import pallas as pl
from jax.experimental.pallas import tpu as pltpu
from jax.experimental.pallas import tpu_sc as plsc

NC = 2
NS = 16
CHUNK = 128
G_SEGS = 64


def _mesh():
    return plsc.VectorSubcoreMesh(
        core_axis_name="c", subcore_axis_name="s", num_cores=NC, num_subcores=NS
    )


def _make_deg(n_pad, e_pad):
    eps = e_pad // (NC * NS)
    n_chunks = eps // CHUNK
    rows_per = n_pad // NS

    @functools.partial(
        pl.kernel,
        out_type=jax.ShapeDtypeStruct((NC * n_pad, 128), jnp.float32),
        mesh=_mesh(),
        scratch_types=[
            pltpu.VMEM_SHARED((n_pad, 128), jnp.float32),
            pltpu.VMEM((CHUNK, 128), jnp.float32),
            pltpu.VMEM((CHUNK,), jnp.int32),
        ],
    )
    def deg_kernel(dst_hbm, z_hbm, out_hbm, acc, ones_v, didx):
        c = lax.axis_index("c")
        s = lax.axis_index("s")
        row0 = s * rows_per
        pltpu.sync_copy(z_hbm.at[pl.ds(row0, rows_per)],
                        acc.at[pl.ds(row0, rows_per)])
        one = jnp.ones((16,), jnp.float32)
        for j in range(CHUNK):
            for k in range(8):
                ones_v[j, pl.ds(16 * k, 16)] = one
        plsc.subcore_barrier()
        ebase = (c * NS + s) * eps

        def body(g, carry):
            base = pl.multiple_of(ebase + g * CHUNK, 8)
            pltpu.sync_copy(dst_hbm.at[pl.ds(base, CHUNK)], didx)
            pltpu.sync_copy(ones_v, acc.at[didx], add=True)
            return carry

        lax.fori_loop(0, n_chunks, body, 0)
        plsc.subcore_barrier()
        pltpu.sync_copy(acc.at[pl.ds(row0, rows_per)],
                        out_hbm.at[pl.ds(c * n_pad + row0, rows_per)])

    return deg_kernel


def _make_agg(n_pad, e_pad, feature_split):
    rows_per = n_pad // NS

    if feature_split:
        eps = e_pad // NS
    else:
        eps = e_pad // (NC * NS)
    n_chunks = eps // CHUNK

    @functools.partial(
        pl.kernel,
        out_type=jax.ShapeDtypeStruct((NC * n_pad, 128), jnp.float32),
        mesh=_mesh(),
        scratch_types=[
            pltpu.VMEM_SHARED((n_pad, 128), jnp.float32),
            pltpu.VMEM((CHUNK, 128), jnp.float32),
            pltpu.VMEM((CHUNK,), jnp.int32),
            pltpu.VMEM((CHUNK,), jnp.int32),
        ],
    )
    def agg_kernel(y_hbm, src_hbm, dst_hbm, z_hbm, out_hbm,
                   acc, rows_v, sidx, didx):
        c = lax.axis_index("c")
        s = lax.axis_index("s")
        row0 = s * rows_per
        pltpu.sync_copy(z_hbm.at[pl.ds(row0, rows_per)],
                        acc.at[pl.ds(row0, rows_per)])
        plsc.subcore_barrier()
        if feature_split:
            sbase = (c * NS + s) * eps
            dbase = s * eps
        else:
            sbase = (c * NS + s) * eps
            dbase = sbase

        def body(g, carry):
            sb = pl.multiple_of(sbase + g * CHUNK, 8)
            db = pl.multiple_of(dbase + g * CHUNK, 8)
            pltpu.sync_copy(src_hbm.at[pl.ds(sb, CHUNK)], sidx)
            pltpu.sync_copy(y_hbm.at[sidx], rows_v)
            pltpu.sync_copy(dst_hbm.at[pl.ds(db, CHUNK)], didx)
            pltpu.sync_copy(rows_v, acc.at[didx], add=True)
            return carry

        lax.fori_loop(0, n_chunks, body, 0)
        plsc.subcore_barrier()
        pltpu.sync_copy(acc.at[pl.ds(row0, rows_per)],
                        out_hbm.at[pl.ds(c * n_pad + row0, rows_per)])

    return agg_kernel


def _k1_body(c0_ref, c1_ref, x_ref, w1_ref, y1_ref, dinv_ref):
    deg = 1.0 + c0_ref[...] + c1_ref[...]
    dinv = lax.rsqrt(deg)
    dinv_ref[...] = dinv
    y1_ref[...] = jnp.dot(x_ref[...], w1_ref[...],
                          preferred_element_type=jnp.float32,
                          precision=lax.Precision.HIGHEST) * dinv


def _k3_body(a0_ref, a1_ref, y1_ref, dinv_ref, b1_ref, w2a_ref, w2b_ref,
             y2a_ref, y2b_ref):
    dinv = dinv_ref[...]
    h = jnp.maximum(dinv * (a0_ref[...] + a1_ref[...] + y1_ref[...])
                    + b1_ref[...], 0.0)
    y2a_ref[...] = jnp.dot(h, w2a_ref[...],
                           preferred_element_type=jnp.float32,
                          precision=lax.Precision.HIGHEST) * dinv
    y2b_ref[...] = jnp.dot(h, w2b_ref[...],
                           preferred_element_type=jnp.float32,
                          precision=lax.Precision.HIGHEST) * dinv


def _k5_body(a2a_ref, a2b_ref, y2a_ref, y2b_ref, dinv_ref, b2a_ref, b2b_ref,
             batch_ref, wba_ref, wbb_ref, bb_ref, out_ref):
    dinv = dinv_ref[...]
    h2a = dinv * (a2a_ref[...] + y2a_ref[...]) + b2a_ref[...]
    h2b = dinv * (a2b_ref[...] + y2b_ref[...]) + b2b_ref[...]
    n_pad = h2a.shape[0]
    seg = lax.broadcasted_iota(jnp.int32, (G_SEGS, n_pad), 0)
    onehot = (batch_ref[...] == seg).astype(jnp.float32)
    sums_a = jnp.dot(onehot, h2a, preferred_element_type=jnp.float32,
                          precision=lax.Precision.HIGHEST)
    sums_b = jnp.dot(onehot, h2b, preferred_element_type=jnp.float32,
                          precision=lax.Precision.HIGHEST)
    counts = jnp.sum(onehot, axis=1, keepdims=True)
    inv_cnt = 1.0 / jnp.maximum(counts, 1.0)
    ea = sums_a * inv_cnt
    eb = sums_b * inv_cnt
    nrm = jnp.sqrt(jnp.sum(ea * ea, axis=1, keepdims=True)
                   + jnp.sum(eb * eb, axis=1, keepdims=True))
    scl = 1.0 / jnp.maximum(nrm, 1e-12)
    out_ref[...] = (jnp.dot(ea * scl, wba_ref[...],
                            preferred_element_type=jnp.float32,
                          precision=lax.Precision.HIGHEST)
                    + jnp.dot(eb * scl, wbb_ref[...],
                              preferred_element_type=jnp.float32,
                          precision=lax.Precision.HIGHEST)
                    + bb_ref[...])


@functools.lru_cache(maxsize=4)
def _build(n, e, f_in, h1, d_emb):
    n_pad = ((n + 1 + 127) // 128) * 128
    e_pad = ((e + NC * NS * CHUNK - 1) // (NC * NS * CHUNK)) * (NC * NS * CHUNK)

    deg_k = _make_deg(n_pad, e_pad)
    agg_edge_k = _make_agg(n_pad, e_pad, feature_split=False)
    agg_feat_k = _make_agg(n_pad, e_pad, feature_split=True)

    k1 = pl.pallas_call(
        _k1_body,
        out_shape=[
            jax.ShapeDtypeStruct((n_pad, h1), jnp.float32),
            jax.ShapeDtypeStruct((n_pad, 1), jnp.float32),
        ],
    )
    k3 = pl.pallas_call(
        _k3_body,
        out_shape=[
            jax.ShapeDtypeStruct((n_pad, 128), jnp.float32),
            jax.ShapeDtypeStruct((n_pad, 128), jnp.float32),
        ],
    )
    k5 = pl.pallas_call(
        _k5_body,
        out_shape=jax.ShapeDtypeStruct((G_SEGS, 2), jnp.float32),
    )
    return n_pad, e_pad, deg_k, agg_edge_k, agg_feat_k, k1, k3, k5


def kernel(x, edge_index, batch, W1, b1, W2, b2, Wb, bb):
    n, f_in = x.shape
    e = edge_index.shape[1]
    h1 = W1.shape[1]
    d_emb = W2.shape[1]
    (n_pad, e_pad, deg_k, agg_edge_k, agg_feat_k, k1, k3, k5) = _build(
        n, e, f_in, h1, d_emb)

    src = edge_index[0]
    dst = edge_index[1]
    e_fill = jnp.full((e_pad - e,), n, jnp.int32)
    src_p = jnp.concatenate([src, e_fill])
    dst_p = jnp.concatenate([dst, e_fill])
    src2 = jnp.concatenate([src_p, src_p + n_pad])
    x_p = jnp.zeros((n_pad, f_in), jnp.float32).at[:n].set(x)
    batch_p = jnp.concatenate(
        [batch, jnp.full((n_pad - n,), G_SEGS, jnp.int32)]).reshape(1, n_pad)
    z128 = jnp.zeros((n_pad, 128), jnp.float32)

    cnt = deg_k(dst_p, z128)
    c0 = cnt[:n_pad, :1]
    c1 = cnt[n_pad:, :1]
    y1, dinv = k1(c0, c1, x_p, W1)
    agg1 = agg_edge_k(y1, src_p, dst_p, z128)
    y2a, y2b = k3(agg1[:n_pad], agg1[n_pad:], y1, dinv,
                  b1.reshape(1, h1), W2[:, :128], W2[:, 128:])
    y2s = jnp.concatenate([y2a, y2b], axis=0)
    agg2 = agg_feat_k(y2s, src2, dst_p, z128)
    out = k5(agg2[:n_pad], agg2[n_pad:], y2a, y2b, dinv,
             b2[:128].reshape(1, 128), b2[128:].reshape(1, 128),
             batch_p, Wb[:128], Wb[128:], bb.reshape(1, 2))
    return out
